# baseline (speedup 1.0000x reference)
import functools

import jax
import jax.numpy as jnp
from jax import lax
from jax.experimental import pallas as pl
from jax.experimental.pallas import tpu as pltpu
from jax.experimental.pallas import tpu_sc as plsc

BATCH = 4096
HIST_LEN = 200
EMBED_DIM = 64
NC = 2
NS = 16
NW = NC * NS
ROWS_PW = BATCH // NW
RPC = 2
N_CHUNKS = ROWS_PW // RPC
SUB0 = 128
SUB1 = HIST_LEN - SUB0


def _make_kernel():
    mesh = plsc.VectorSubcoreMesh(core_axis_name="c", subcore_axis_name="s")

    @functools.partial(
        pl.kernel,
        out_type=jax.ShapeDtypeStruct((BATCH, HIST_LEN, EMBED_DIM), jnp.float32),
        mesh=mesh,
        scratch_types=[
            pltpu.VMEM((ROWS_PW, HIST_LEN), jnp.int32),
            pltpu.VMEM((2, RPC, HIST_LEN, EMBED_DIM), jnp.float32),
            pltpu.SemaphoreType.DMA,
            pltpu.SemaphoreType.DMA,
            pltpu.SemaphoreType.DMA,
        ],
        compiler_params=pltpu.CompilerParams(use_tc_tiling_on_sc=False),
    )
    def gather_kernel(table_hbm, q_hbm, out_hbm, idx_v, rows_v, gsem, wsem0, wsem1):
        wid = lax.axis_index("s") * NC + lax.axis_index("c")
        base_row = wid * ROWS_PW
        pltpu.sync_copy(q_hbm.at[pl.ds(base_row, ROWS_PW)], idx_v)

        wsems = (wsem0, wsem1)

        def do_chunk(c, b, first):
            wb = pltpu.make_async_copy(
                rows_v.at[b],
                out_hbm.at[pl.ds(base_row + c * RPC, RPC)],
                wsems[b],
            )
            if not first:
                wb.wait()
            cps = []
            for rr in range(RPC):
                r = c * RPC + rr
                cps.append(pltpu.async_copy(
                    table_hbm.at[idx_v.at[r, pl.ds(0, SUB0)]],
                    rows_v.at[b, rr, pl.ds(0, SUB0)],
                    gsem,
                ))
                cps.append(pltpu.async_copy(
                    table_hbm.at[idx_v.at[r, pl.ds(SUB0, SUB1)]],
                    rows_v.at[b, rr, pl.ds(SUB0, SUB1)],
                    gsem,
                ))
            for cp in cps:
                cp.wait()
            wb.start()

        def pair_body(p, carry):
            for b in range(2):
                do_chunk(p * 2 + b, b, first=False)
            return carry

        for b in range(2):
            do_chunk(b, b, first=True)
        lax.fori_loop(1, N_CHUNKS // 2, pair_body, 0)
        for b in range(2):
            pltpu.make_async_copy(
                rows_v.at[b],
                out_hbm.at[pl.ds(base_row, RPC)],
                wsems[b],
            ).wait()

    return gather_kernel


_gather = _make_kernel()


@jax.jit
def kernel(questions, embedding_weights):
    return _gather(embedding_weights, questions.astype(jnp.int32))


# speedup vs baseline: 1.0040x; 1.0040x over previous
import functools

import jax
import jax.numpy as jnp
from jax import lax
from jax.experimental import pallas as pl
from jax.experimental.pallas import tpu as pltpu
from jax.experimental.pallas import tpu_sc as plsc

BATCH = 4096
HIST_LEN = 200
VOCAB = 1000000
EMBED_DIM = 64
NC = 2
NS = 16
NW = NC * NS
ROWS_PW = BATCH // NW
RPC = 4
N_CHUNKS = ROWS_PW // RPC
SUB0 = 128
SUB1 = HIST_LEN - SUB0

def _make_gather():
    mesh = plsc.VectorSubcoreMesh(core_axis_name="c", subcore_axis_name="s")

    @functools.partial(
        pl.kernel,
        out_type=jax.ShapeDtypeStruct((BATCH, HIST_LEN, EMBED_DIM), jnp.float32),
        mesh=mesh,
        scratch_types=[
            pltpu.VMEM((ROWS_PW, HIST_LEN), jnp.int32),
            pltpu.VMEM((2, RPC, HIST_LEN, EMBED_DIM), jnp.float32),
            pltpu.SemaphoreType.DMA,
            pltpu.SemaphoreType.DMA,
            pltpu.SemaphoreType.DMA,
        ],
        compiler_params=pltpu.CompilerParams(use_tc_tiling_on_sc=False),
    )
    def gather_kernel(table_hbm, q_hbm, out_hbm, idx_v, rows_v, gsem, wsem0, wsem1):
        wid = lax.axis_index("s") * NC + lax.axis_index("c")
        base_row = wid * ROWS_PW
        pltpu.sync_copy(q_hbm.at[pl.ds(base_row, ROWS_PW)], idx_v)

        wsems = (wsem0, wsem1)

        def do_chunk(c, b, first):
            wb = pltpu.make_async_copy(
                rows_v.at[b],
                out_hbm.at[pl.ds(base_row + c * RPC, RPC)],
                wsems[b],
            )
            if not first:
                wb.wait()
            cps = []
            for rr in range(RPC):
                r = c * RPC + rr
                cps.append(pltpu.async_copy(
                    table_hbm.at[idx_v.at[r, pl.ds(0, SUB0)]],
                    rows_v.at[b, rr, pl.ds(0, SUB0)],
                    gsem,
                ))
                cps.append(pltpu.async_copy(
                    table_hbm.at[idx_v.at[r, pl.ds(SUB0, SUB1)]],
                    rows_v.at[b, rr, pl.ds(SUB0, SUB1)],
                    gsem,
                ))
            for cp in cps:
                cp.wait()
            wb.start()

        def pair_body(p, carry):
            for b in range(2):
                do_chunk(p * 2 + b, b, first=False)
            return carry

        for b in range(2):
            do_chunk(b, b, first=True)
        lax.fori_loop(1, N_CHUNKS // 2, pair_body, 0)
        for b in range(2):
            pltpu.make_async_copy(
                rows_v.at[b],
                out_hbm.at[pl.ds(base_row, RPC)],
                wsems[b],
            ).wait()

    return gather_kernel


_gather = _make_gather()


@jax.jit
def kernel(questions, embedding_weights):
    return _gather(embedding_weights, questions.astype(jnp.int32))
